# baseline (speedup 1.0000x reference)
import functools
import math

import jax
import jax.numpy as jnp
from jax import lax
from jax.experimental import pallas as pl
from jax.experimental.pallas import tpu as pltpu
from jax.experimental.pallas import tpu_sc as plsc

T, D, H, E = 2048, 768, 3072, 64
CAP = 80
SLAB = 2 * CAP
NROWS = E * SLAB
DUMP = NROWS
NPAD = NROWS + 8
HHALF = H // 2
NW = 32
PERW = 2 * T // NW
F32 = jnp.float32
BF16 = jnp.bfloat16
I32 = jnp.int32


def _onehot_chunk(c, width):
    r = lax.broadcasted_iota(I32, (T, width), 0)
    col = lax.broadcasted_iota(I32, (T, width), 1)
    return (r == c * width + col).astype(BF16)


def _col_to_row_i32(vcol):
    planes = []
    for k in range(4):
        byte = lax.shift_right_logical(vcol, 8 * k) & 0xFF
        bf = byte.astype(F32).astype(BF16)
        rows = []
        for c in range(4):
            oh = _onehot_chunk(c, 512)
            rows.append(lax.dot_general(bf, oh, (((0,), (0,)), ((), ())),
                                        preferred_element_type=F32))
        row = jnp.concatenate(rows, axis=1)
        planes.append(lax.shift_left(row.astype(I32), 8 * k))
    return planes[0] | planes[1] | planes[2] | planes[3]


def _col_to_row_f32(vcol):
    return lax.bitcast_convert_type(
        _col_to_row_i32(lax.bitcast_convert_type(vcol, I32)), F32)


def _router_body(x_ref, wr_ref, aux_ref, w_ref, g_ref):
    xx = x_ref[...]
    wr = wr_ref[...]
    logits = lax.dot_general(xx.astype(BF16), wr.astype(BF16),
                             (((1,), (1,)), ((), ())),
                             preferred_element_type=F32)
    mx = jnp.max(logits, axis=1, keepdims=True)
    pexp = jnp.exp(logits - mx)
    probs = pexp / jnp.sum(pexp, axis=1, keepdims=True)
    imp = jnp.sum(probs, axis=0, keepdims=True) * (1.0 / T)

    ie = lax.broadcasted_iota(I32, (1, E), 1)
    v1 = jnp.max(probs, axis=1, keepdims=True)
    i1 = jnp.min(jnp.where(probs == v1, ie, E), axis=1, keepdims=True)
    probs2 = jnp.where(ie == i1, -1.0, probs)
    v2 = jnp.max(probs2, axis=1, keepdims=True)
    i2 = jnp.min(jnp.where(probs2 == v2, ie, E), axis=1, keepdims=True)
    ssum = v1 + v2 + 1e-8
    w0 = v1 / ssum
    w1 = v2 / ssum

    i1row = _col_to_row_i32(i1)
    i2row = _col_to_row_i32(i2)
    w0row = _col_to_row_f32(w0)
    w1row = _col_to_row_f32(w1)
    tcol = lax.broadcasted_iota(I32, (T, 1), 0)

    def rank_of(icol, wcol, irow, wrow):
        cnt = jnp.zeros((T, 1), F32)
        for c in range(4):
            sl = slice(c * 512, (c + 1) * 512)
            ir = irow[:, sl]
            wr_ = wrow[:, sl]
            tp = lax.broadcasted_iota(I32, (T, 512), 1) + c * 512
            same = ir == icol
            gt = (wr_ > wcol) | ((wr_ == wcol) & (tp < tcol))
            cnt = cnt + jnp.sum((same & gt).astype(F32), axis=1, keepdims=True)
        return cnt

    r0 = rank_of(i1, w0, i1row, w0row)
    r1 = rank_of(i2, w1, i2row, w1row)
    drop = (r0 >= CAP) | (r1 >= CAP)
    w0f = jnp.where(drop, 0.0, w0)
    w1f = jnp.where(drop, 0.0, w1)
    rs = jnp.maximum(w0f + w1f, 1e-8)
    w0f = w0f / rs
    w1f = w1f / rs

    keep = jnp.logical_not(drop)
    iecast = lax.broadcasted_iota(I32, (T, E), 1)
    m0 = ((i1 == iecast) & keep).astype(BF16)
    m1 = ((i2 == iecast) & keep).astype(BF16)

    mcat = jnp.concatenate([m0, m1], axis=1)
    pos_chunks = []
    for c in range(8):
        rsub = lax.broadcasted_iota(I32, (256, T), 0) + c * 256
        tlane = lax.broadcasted_iota(I32, (256, T), 1)
        tri = (tlane < rsub).astype(BF16)
        pos_chunks.append(lax.dot_general(tri, mcat, (((1,), (0,)), ((), ())),
                                          preferred_element_type=F32))
    pos = jnp.concatenate(pos_chunks, axis=0)
    pos0, pos1 = pos[:, :E], pos[:, E:]
    m0f = m0.astype(F32)
    m1f = m1.astype(F32)
    c0 = jnp.sum(m0f, axis=0, keepdims=True)
    pos0t = jnp.sum(m0f * pos0, axis=1, keepdims=True)
    pos1t = jnp.sum(m1f * (pos1 + c0), axis=1, keepdims=True)
    g0 = jnp.where(keep, i1 * SLAB + pos0t.astype(I32), DUMP)
    g1 = jnp.where(keep, i2 * SLAB + pos1t.astype(I32), DUMP)

    load = jnp.sum(m0f * w0f + m1f * w1f, axis=0, keepdims=True) * (1.0 / T)
    aux_ref[...] = jnp.sum(imp * load * float(E), axis=1, keepdims=True)
    w_ref[...] = jnp.concatenate([w0f, w1f], axis=1)
    g_ref[...] = jnp.concatenate([g0, g1], axis=1)


def _k1(x2d, wr, interpret=False):
    return pl.pallas_call(
        _router_body,
        out_shape=(
            jax.ShapeDtypeStruct((1, 1), F32),
            jax.ShapeDtypeStruct((T, 2), F32),
            jax.ShapeDtypeStruct((T, 2), I32),
        ),
        interpret=interpret,
    )(x2d, wr)


def _ffn_body(disp_ref, w1_ref, b1_ref, w2_ref, b2_ref, y_ref):
    g = disp_ref[...].astype(BF16)
    h = lax.dot_general(g, w1_ref[0].astype(BF16), (((1,), (0,)), ((), ())),
                        preferred_element_type=F32)
    h = h + b1_ref[0]
    h = h * 0.5 * (1.0 + lax.erf(h * (1.0 / math.sqrt(2.0))))
    y = lax.dot_general(h.astype(BF16), w2_ref[0].astype(BF16),
                        (((1,), (0,)), ((), ())), preferred_element_type=F32)
    y_ref[...] = y + b2_ref[0]


def _k2(disp, w1, b1r, w2, b2r, interpret=False):
    return pl.pallas_call(
        _ffn_body,
        grid=(E,),
        in_specs=[
            pl.BlockSpec((SLAB, D), lambda e: (e, 0)),
            pl.BlockSpec((1, D, H), lambda e: (e, 0, 0)),
            pl.BlockSpec((1, 1, H), lambda e: (e, 0, 0)),
            pl.BlockSpec((1, H, D), lambda e: (e, 0, 0)),
            pl.BlockSpec((1, 1, D), lambda e: (e, 0, 0)),
        ],
        out_specs=pl.BlockSpec((SLAB, D), lambda e: (e, 0)),
        out_shape=jax.ShapeDtypeStruct((NPAD, D), F32),
        interpret=interpret,
    )(disp, w1, b1r, w2, b2r)


def _combine_body(g0_ref, g1_ref, w_ref, o_ref):
    w0 = w_ref[:, 0:1]
    w1 = w_ref[:, 1:2]
    a = jnp.where(w0 > 0, w0 * g0_ref[...], 0.0)
    b = jnp.where(w1 > 0, w1 * g1_ref[...], 0.0)
    o_ref[...] = a + b


def _k3(gath, wcat, interpret=False):
    nblk = 8
    tb = T // nblk
    return pl.pallas_call(
        _combine_body,
        grid=(nblk,),
        in_specs=[
            pl.BlockSpec((tb, D), lambda i: (i, 0)),
            pl.BlockSpec((tb, D), lambda i: (nblk + i, 0)),
            pl.BlockSpec((tb, 2), lambda i: (i, 0)),
        ],
        out_specs=pl.BlockSpec((tb, D), lambda i: (i, 0)),
        out_shape=jax.ShapeDtypeStruct((T, D), F32),
        interpret=interpret,
    )(gath, gath, wcat)


def _s1(x2d, gidx32):
    mesh = plsc.VectorSubcoreMesh(core_axis_name="c", subcore_axis_name="s")

    @functools.partial(
        pl.kernel, mesh=mesh,
        out_type=jax.ShapeDtypeStruct((NPAD, D), F32),
        scratch_types=[
            pltpu.VMEM((PERW,), I32),
            pltpu.VMEM((PERW, D), F32),
        ],
    )
    def body(x_hbm, g_hbm, disp_hbm, idx_v, rows_v):
        wid = lax.axis_index("s") * 2 + lax.axis_index("c")
        tok = lax.rem(wid * PERW, T)
        pltpu.sync_copy(x_hbm.at[pl.ds(tok, PERW)], rows_v)
        pltpu.sync_copy(g_hbm.at[wid], idx_v)
        pltpu.sync_copy(rows_v, disp_hbm.at[idx_v])

    return body(x2d, gidx32)


def _s2(y, gidx32):
    mesh = plsc.VectorSubcoreMesh(core_axis_name="c", subcore_axis_name="s")

    @functools.partial(
        pl.kernel, mesh=mesh,
        out_type=jax.ShapeDtypeStruct((2 * T, D), F32),
        scratch_types=[
            pltpu.VMEM((PERW,), I32),
            pltpu.VMEM((PERW, D), F32),
        ],
    )
    def body(y_hbm, g_hbm, gath_hbm, idx_v, rows_v):
        wid = lax.axis_index("s") * 2 + lax.axis_index("c")
        pltpu.sync_copy(g_hbm.at[wid], idx_v)
        pltpu.sync_copy(y_hbm.at[idx_v], rows_v)
        pltpu.sync_copy(rows_v, gath_hbm.at[pl.ds(wid * PERW, PERW)])

    return body(y, gidx32)


def kernel(x, Wr, W1, b1, W2, b2):
    x2d = x.reshape(T, D)
    aux11, wcat, gcat = _k1(x2d, Wr)
    gidx32 = jnp.concatenate([gcat[:, 0], gcat[:, 1]], axis=0).reshape(NW, PERW)
    disp = _s1(x2d, gidx32)
    y = _k2(disp, W1, b1.reshape(E, 1, H), W2, b2.reshape(E, 1, D))
    gath = _s2(y, gidx32)
    out = _k3(gath, wcat)
    return out.reshape(1, T, D), aux11.reshape(())


# speedup vs baseline: 1.0706x; 1.0706x over previous
import functools
import math

import jax
import jax.numpy as jnp
from jax import lax
from jax.experimental import pallas as pl
from jax.experimental.pallas import tpu as pltpu
from jax.experimental.pallas import tpu_sc as plsc

T, D, H, E = 2048, 768, 3072, 64
CAP = 80
SLAB = 2 * CAP
NROWS = E * SLAB
DUMP = NROWS
NPAD = NROWS + 8
HHALF = H // 2
D2 = D // 2
NW = 32
PERW = 2 * T // NW
F32 = jnp.float32
BF16 = jnp.bfloat16
I32 = jnp.int32


U32 = jnp.uint32


def _pack_bf16(a):
    h = a.shape[-1] // 2
    af = a.astype(BF16).astype(F32)
    lo = lax.bitcast_convert_type(af[..., :h], U32)
    hi = lax.bitcast_convert_type(af[..., h:], U32)
    u = jnp.right_shift(lo, 16) | jnp.left_shift(jnp.right_shift(hi, 16), 16)
    return lax.bitcast_convert_type(u, F32)


def _unpack_bf16(a):
    u = lax.bitcast_convert_type(a, U32)
    lo = lax.bitcast_convert_type(jnp.left_shift(u, 16), F32)
    hi = lax.bitcast_convert_type(jnp.left_shift(jnp.right_shift(u, 16), 16), F32)
    return jnp.concatenate([lo, hi], axis=-1).astype(BF16)


def _router_body(x_ref, wr_ref, aux_ref, w_ref, g_ref, xb_ref, drop_ref):
    xx = x_ref[...]
    wr = wr_ref[...]
    xb16 = xx.astype(BF16)
    xb_ref[...] = _pack_bf16(xx)
    logits = lax.dot_general(xb16, wr.astype(BF16),
                             (((1,), (1,)), ((), ())),
                             preferred_element_type=F32)
    mx = jnp.max(logits, axis=1, keepdims=True)
    pexp = jnp.exp(logits - mx)
    probs = pexp / jnp.sum(pexp, axis=1, keepdims=True)
    imp = jnp.sum(probs, axis=0, keepdims=True) * (1.0 / T)

    ie = lax.broadcasted_iota(I32, (1, E), 1)
    v1 = jnp.max(probs, axis=1, keepdims=True)
    i1 = jnp.min(jnp.where(probs == v1, ie, E), axis=1, keepdims=True)
    probs2 = jnp.where(ie == i1, -1.0, probs)
    v2 = jnp.max(probs2, axis=1, keepdims=True)
    i2 = jnp.min(jnp.where(probs2 == v2, ie, E), axis=1, keepdims=True)
    ssum = v1 + v2 + 1e-8
    w0 = v1 / ssum
    w1 = v2 / ssum

    iecast = lax.broadcasted_iota(I32, (T, E), 1)
    oh1 = (i1 == iecast)
    oh2 = (i2 == iecast)
    cnt_full = jnp.sum(oh1.astype(F32), axis=0, keepdims=True)
    cnt_full2 = jnp.sum(oh2.astype(F32), axis=0, keepdims=True)
    need_rank = jnp.maximum(jnp.max(cnt_full), jnp.max(cnt_full2)) > CAP

    drop_ref[...] = jnp.zeros((T, 1), F32)

    @pl.when(need_rank)
    def compute_drop():
        w0b = lax.bitcast_convert_type(w0, I32)
        w1b = lax.bitcast_convert_type(w1, I32)
        planes = [i1.astype(F32), i2.astype(F32)]
        for src in (w0b, w1b):
            for k in range(4):
                planes.append(
                    (lax.shift_right_logical(src, 8 * k) & 0xFF).astype(F32))
        pmat = jnp.concatenate(planes, axis=1).astype(BF16)
        rows = []
        for c in range(4):
            r = lax.broadcasted_iota(I32, (T, 512), 0)
            col = lax.broadcasted_iota(I32, (T, 512), 1)
            oh = (r == c * 512 + col).astype(BF16)
            rows.append(lax.dot_general(pmat, oh, (((0,), (0,)), ((), ())),
                                        preferred_element_type=F32))
        prow = jnp.concatenate(rows, axis=1)
        i1row = prow[0:1].astype(I32)
        i2row = prow[1:2].astype(I32)

        def unbytes(off):
            acc = prow[off:off + 1].astype(I32)
            for k in range(1, 4):
                acc = acc | lax.shift_left(prow[off + k:off + k + 1].astype(I32), 8 * k)
            return lax.bitcast_convert_type(acc, F32)

        w0row = unbytes(2)
        w1row = unbytes(6)
        tcol = lax.broadcasted_iota(I32, (T, 1), 0)

        def rank_of(icol, wcol, irow, wrow):
            cnt = jnp.zeros((T, 1), F32)
            for c in range(4):
                sl = slice(c * 512, (c + 1) * 512)
                ir = irow[:, sl]
                wr_ = wrow[:, sl]
                tp = lax.broadcasted_iota(I32, (T, 512), 1) + c * 512
                same = ir == icol
                gt = (wr_ > wcol) | ((wr_ == wcol) & (tp < tcol))
                cnt = cnt + jnp.sum((same & gt).astype(F32), axis=1,
                                    keepdims=True)
            return cnt

        r0 = rank_of(i1, w0, i1row, w0row)
        r1 = rank_of(i2, w1, i2row, w1row)
        drop_ref[...] = ((r0 >= CAP) | (r1 >= CAP)).astype(F32)

    drop = drop_ref[...] > 0.0
    w0f = jnp.where(drop, 0.0, w0)
    w1f = jnp.where(drop, 0.0, w1)
    rs = jnp.maximum(w0f + w1f, 1e-8)
    w0f = w0f / rs
    w1f = w1f / rs

    keep = jnp.logical_not(drop)
    m0 = (oh1 & keep).astype(BF16)
    m1 = (oh2 & keep).astype(BF16)

    mcat = jnp.concatenate([m0, m1], axis=1)
    pos_chunks = []
    for c in range(8):
        rsub = lax.broadcasted_iota(I32, (256, T), 0) + c * 256
        tlane = lax.broadcasted_iota(I32, (256, T), 1)
        tri = (tlane < rsub).astype(BF16)
        pos_chunks.append(lax.dot_general(tri, mcat, (((1,), (0,)), ((), ())),
                                          preferred_element_type=F32))
    pos = jnp.concatenate(pos_chunks, axis=0)
    pos0, pos1 = pos[:, :E], pos[:, E:]
    m0f = m0.astype(F32)
    m1f = m1.astype(F32)
    c0 = jnp.sum(m0f, axis=0, keepdims=True)
    pos0t = jnp.sum(m0f * pos0, axis=1, keepdims=True)
    pos1t = jnp.sum(m1f * (pos1 + c0), axis=1, keepdims=True)
    g0 = jnp.where(keep, i1 * SLAB + pos0t.astype(I32), DUMP)
    g1 = jnp.where(keep, i2 * SLAB + pos1t.astype(I32), DUMP)

    load = jnp.sum(m0f * w0f + m1f * w1f, axis=0, keepdims=True) * (1.0 / T)
    aux_ref[...] = jnp.sum(imp * load * float(E), axis=1, keepdims=True)
    w_ref[...] = jnp.concatenate([w0f, w1f], axis=1)
    g_ref[...] = jnp.concatenate([g0, g1], axis=1)


def _k1(x2d, wr, interpret=False):
    return pl.pallas_call(
        _router_body,
        out_shape=(
            jax.ShapeDtypeStruct((1, 1), F32),
            jax.ShapeDtypeStruct((T, 2), F32),
            jax.ShapeDtypeStruct((T, 2), I32),
            jax.ShapeDtypeStruct((T, D2), F32),
        ),
        scratch_shapes=[pltpu.VMEM((T, 1), F32)],
        interpret=interpret,
    )(x2d, wr)


def _ffn_body(disp_ref, w1_ref, b1_ref, w2_ref, b2_ref, y_ref, acc_ref):
    j = pl.program_id(1)
    g = _unpack_bf16(disp_ref[...])
    h = lax.dot_general(g, w1_ref[0].astype(BF16), (((1,), (0,)), ((), ())),
                        preferred_element_type=F32)
    h = h + b1_ref[0]
    h = h * 0.5 * (1.0 + lax.erf(h * (1.0 / math.sqrt(2.0))))
    y = lax.dot_general(h.astype(BF16), w2_ref[0].astype(BF16),
                        (((1,), (0,)), ((), ())), preferred_element_type=F32)

    @pl.when(j == 0)
    def _():
        acc_ref[...] = y

    @pl.when(j == 1)
    def _():
        y_ref[...] = _pack_bf16(acc_ref[...] + y + b2_ref[0])


def _k2(disp, w1, b1r, w2, b2r, interpret=False):
    return pl.pallas_call(
        _ffn_body,
        grid=(E, 2),
        in_specs=[
            pl.BlockSpec((SLAB, D2), lambda e, j: (e, 0)),
            pl.BlockSpec((1, D, HHALF), lambda e, j: (e, 0, j)),
            pl.BlockSpec((1, 1, HHALF), lambda e, j: (e, 0, j)),
            pl.BlockSpec((1, HHALF, D), lambda e, j: (e, j, 0)),
            pl.BlockSpec((1, 1, D), lambda e, j: (e, 0, 0)),
        ],
        out_specs=pl.BlockSpec((SLAB, D2), lambda e, j: (e, 0)),
        out_shape=jax.ShapeDtypeStruct((NPAD, D2), F32),
        scratch_shapes=[pltpu.VMEM((SLAB, D), F32)],
        interpret=interpret,
    )(disp, w1, b1r, w2, b2r)


def _combine_body(g0_ref, g1_ref, w_ref, o_ref):
    w0 = w_ref[:, 0:1]
    w1 = w_ref[:, 1:2]
    a = jnp.where(w0 > 0, w0 * _unpack_bf16(g0_ref[...]).astype(F32), 0.0)
    b = jnp.where(w1 > 0, w1 * _unpack_bf16(g1_ref[...]).astype(F32), 0.0)
    o_ref[...] = a + b


def _k3(gath, wcat, interpret=False):
    nblk = 8
    tb = T // nblk
    return pl.pallas_call(
        _combine_body,
        grid=(nblk,),
        in_specs=[
            pl.BlockSpec((tb, D2), lambda i: (i, 0)),
            pl.BlockSpec((tb, D2), lambda i: (nblk + i, 0)),
            pl.BlockSpec((tb, 2), lambda i: (i, 0)),
        ],
        out_specs=pl.BlockSpec((tb, D), lambda i: (i, 0)),
        out_shape=jax.ShapeDtypeStruct((T, D), F32),
        interpret=interpret,
    )(gath, gath, wcat)


def _s1(xb, gidx32):
    mesh = plsc.VectorSubcoreMesh(core_axis_name="c", subcore_axis_name="s")

    @functools.partial(
        pl.kernel, mesh=mesh,
        out_type=jax.ShapeDtypeStruct((NPAD, D2), F32),
        scratch_types=[
            pltpu.VMEM((PERW,), I32),
            pltpu.VMEM((PERW, D2), F32),
        ],
    )
    def body(x_hbm, g_hbm, disp_hbm, idx_v, rows_v):
        wid = lax.axis_index("s") * 2 + lax.axis_index("c")
        tok = lax.rem(wid * PERW, T)
        pltpu.sync_copy(x_hbm.at[pl.ds(tok, PERW)], rows_v)
        pltpu.sync_copy(g_hbm.at[wid], idx_v)
        pltpu.sync_copy(rows_v, disp_hbm.at[idx_v])

    return body(xb, gidx32)


def _s2(y, gidx32):
    mesh = plsc.VectorSubcoreMesh(core_axis_name="c", subcore_axis_name="s")

    @functools.partial(
        pl.kernel, mesh=mesh,
        out_type=jax.ShapeDtypeStruct((2 * T, D2), F32),
        scratch_types=[
            pltpu.VMEM((PERW,), I32),
            pltpu.VMEM((PERW, D2), F32),
        ],
    )
    def body(y_hbm, g_hbm, gath_hbm, idx_v, rows_v):
        wid = lax.axis_index("s") * 2 + lax.axis_index("c")
        pltpu.sync_copy(g_hbm.at[wid], idx_v)
        pltpu.sync_copy(y_hbm.at[idx_v], rows_v)
        pltpu.sync_copy(rows_v, gath_hbm.at[pl.ds(wid * PERW, PERW)])

    return body(y, gidx32)


def kernel(x, Wr, W1, b1, W2, b2):
    x2d = x.reshape(T, D)
    aux11, wcat, gcat, xb = _k1(x2d, Wr)
    gidx32 = jnp.concatenate([gcat[:, 0], gcat[:, 1]], axis=0).reshape(NW, PERW)
    disp = _s1(xb, gidx32)
    y = _k2(disp, W1, b1.reshape(E, 1, H), W2, b2.reshape(E, 1, D))
    gath = _s2(y, gidx32)
    out = _k3(gath, wcat)
    return out.reshape(1, T, D), aux11.reshape(())
